# baseline (speedup 1.0000x reference)
import functools
import jax
import jax.numpy as jnp
from jax import lax
from jax.experimental import pallas as pl
from jax.experimental.pallas import tpu as pltpu
from jax.experimental.pallas import tpu_sc as plsc

D_MODEL = 4096
NUM_EXPERTS = 64
TOP_K = 8
N_TOKENS = 16384
BLK = 1024
GRID = N_TOKENS // BLK

NC = 2
NS = 16
NW = NC * NS
TPW = N_TOKENS // NW
GPW = TPW // 16


def _tc_kernel(u_ref, e_ref, bias_ref, scores_ref, aux_ref, psum_ref):
    i = pl.program_id(0)
    logits = jnp.dot(u_ref[...], e_ref[...],
                     preferred_element_type=jnp.float32) + bias_ref[...]
    scores = jax.nn.sigmoid(logits)
    scores_ref[...] = scores

    e = jnp.exp(scores)
    probs = e / jnp.sum(e, axis=1, keepdims=True)
    col = jnp.sum(probs, axis=0).reshape(1, NUM_EXPERTS)

    @pl.when(i == 0)
    def _init():
        psum_ref[...] = jnp.zeros_like(psum_ref)

    psum_ref[...] += col

    @pl.when(i == GRID - 1)
    def _fin():
        mean = psum_ref[...] / N_TOKENS
        aux_ref[...] = (jnp.sum(mean * mean) * NUM_EXPERTS).reshape(1, 1)


def _tc_scores(u, E, bias):
    bias2 = bias.reshape(1, NUM_EXPERTS)
    out_shape = (
        jax.ShapeDtypeStruct((N_TOKENS, NUM_EXPERTS), jnp.float32),
        jax.ShapeDtypeStruct((1, 1), jnp.float32),
    )
    scores, aux = pl.pallas_call(
        _tc_kernel,
        grid=(GRID,),
        in_specs=[
            pl.BlockSpec((BLK, D_MODEL), lambda i: (i, 0)),
            pl.BlockSpec((D_MODEL, NUM_EXPERTS), lambda i: (0, 0)),
            pl.BlockSpec((1, NUM_EXPERTS), lambda i: (0, 0)),
        ],
        out_specs=(
            pl.BlockSpec((BLK, NUM_EXPERTS), lambda i: (i, 0)),
            pl.BlockSpec((1, 1), lambda i: (0, 0)),
        ),
        out_shape=out_shape,
        scratch_shapes=[pltpu.VMEM((1, NUM_EXPERTS), jnp.float32)],
    )(u, E, bias2)
    return scores, aux


_SC_MESH = plsc.VectorSubcoreMesh(core_axis_name="c", subcore_axis_name="s")


@functools.partial(
    pl.kernel,
    mesh=_SC_MESH,
    compiler_params=pltpu.CompilerParams(needs_layout_passes=False),
    out_type=(
        jax.ShapeDtypeStruct((N_TOKENS * TOP_K,), jnp.int32),
        jax.ShapeDtypeStruct((N_TOKENS * TOP_K,), jnp.float32),
    ),
    scratch_types=[
        pltpu.VMEM((TPW * NUM_EXPERTS,), jnp.float32),
        pltpu.VMEM((NUM_EXPERTS * 16,), jnp.float32),
        pltpu.VMEM((TPW * TOP_K,), jnp.int32),
        pltpu.VMEM((TPW * TOP_K,), jnp.float32),
    ],
)
def _sc_topk(scores_hbm, ti_hbm, ts_hbm, sc_v, xt_v, ti_v, ts_v):
    wid = lax.axis_index("s") * NC + lax.axis_index("c")
    base = wid * TPW
    pltpu.sync_copy(scores_hbm.at[pl.ds(base * NUM_EXPERTS,
                                        TPW * NUM_EXPERTS)], sc_v)

    lane = lax.iota(jnp.int32, 16)
    neg_inf = jnp.full((16,), -jnp.inf, jnp.float32)

    def group_body(g, carry):
        row_base = (g * 16 + lane) * NUM_EXPERTS
        for e in range(NUM_EXPERTS):
            v = plsc.load_gather(sc_v, [row_base + e])
            xt_v[pl.ds(e * 16, 16)] = v

        out_base = (g * 16 + lane) * TOP_K
        for r in range(TOP_K):
            m = neg_inf
            for e in range(NUM_EXPERTS):
                m = jnp.maximum(m, xt_v[pl.ds(e * 16, 16)])
            am = jnp.full((16,), NUM_EXPERTS, jnp.int32)
            for e in range(NUM_EXPERTS - 1, -1, -1):
                v = xt_v[pl.ds(e * 16, 16)]
                am = jnp.where(v == m, jnp.int32(e), am)
            for e in range(NUM_EXPERTS):
                v = xt_v[pl.ds(e * 16, 16)]
                xt_v[pl.ds(e * 16, 16)] = jnp.where(am == e, neg_inf, v)
            plsc.store_scatter(ts_v, [out_base + r], m)
            plsc.store_scatter(ti_v, [out_base + r], am)
        return carry

    lax.fori_loop(0, GPW, group_body, jnp.int32(0))

    pltpu.sync_copy(ti_v, ti_hbm.at[pl.ds(base * TOP_K, TPW * TOP_K)])
    pltpu.sync_copy(ts_v, ts_hbm.at[pl.ds(base * TOP_K, TPW * TOP_K)])


def kernel(u, E, bias):
    scores, aux = _tc_scores(u, E, bias)
    ti_flat, ts_flat = _sc_topk(scores.reshape(-1))
    topk_i = ti_flat.reshape(N_TOKENS, TOP_K)
    topk_s = ts_flat.reshape(N_TOKENS, TOP_K)
    return topk_i, topk_s, scores, aux[0, 0]


# speedup vs baseline: 2.4934x; 2.4934x over previous
import jax
import jax.numpy as jnp
from jax.experimental import pallas as pl
from jax.experimental.pallas import tpu as pltpu

D_MODEL = 4096
NUM_EXPERTS = 64
TOP_K = 8
N_TOKENS = 16384
BLK = 1024
GRID = N_TOKENS // BLK


def _router_kernel(u_ref, e_ref, bias_ref, topk_i_ref, topk_s_ref,
                   scores_ref, aux_ref, psum_ref):
    i = pl.program_id(0)
    logits = jnp.dot(u_ref[...], e_ref[...],
                     preferred_element_type=jnp.float32) + bias_ref[...]
    scores = jax.nn.sigmoid(logits)
    scores_ref[...] = scores

    e = jnp.exp(scores)
    probs = e / jnp.sum(e, axis=1, keepdims=True)
    col = jnp.sum(probs, axis=0).reshape(1, NUM_EXPERTS)

    @pl.when(i == 0)
    def _init():
        psum_ref[...] = jnp.zeros_like(psum_ref)

    psum_ref[...] += col

    xt = scores.T
    iota_t = jax.lax.broadcasted_iota(jnp.int32, (NUM_EXPERTS, BLK),
                                      0).astype(jnp.float32)
    x = xt
    vals = []
    fidxs = []
    for _ in range(TOP_K):
        mx = jnp.max(x, axis=0, keepdims=True)
        idx = jnp.min(jnp.where(x == mx, iota_t, jnp.float32(NUM_EXPERTS)),
                      axis=0, keepdims=True)
        vals.append(mx)
        fidxs.append(idx)
        x = jnp.where(iota_t == idx, -jnp.inf, x)
    topk_s_ref[...] = jnp.concatenate(vals, axis=0)
    topk_i_ref[...] = jnp.concatenate(fidxs, axis=0).astype(jnp.int32)

    @pl.when(i == GRID - 1)
    def _fin():
        mean = psum_ref[...] / N_TOKENS
        aux_ref[...] = (jnp.sum(mean * mean) * NUM_EXPERTS).reshape(1, 1)


def kernel(u, E, bias):
    bias2 = bias.reshape(1, NUM_EXPERTS)
    out_shape = (
        jax.ShapeDtypeStruct((TOP_K, N_TOKENS), jnp.int32),
        jax.ShapeDtypeStruct((TOP_K, N_TOKENS), jnp.float32),
        jax.ShapeDtypeStruct((N_TOKENS, NUM_EXPERTS), jnp.float32),
        jax.ShapeDtypeStruct((1, 1), jnp.float32),
    )
    topk_i_t, topk_s_t, scores, aux = pl.pallas_call(
        _router_kernel,
        grid=(GRID,),
        in_specs=[
            pl.BlockSpec((BLK, D_MODEL), lambda i: (i, 0)),
            pl.BlockSpec((D_MODEL, NUM_EXPERTS), lambda i: (0, 0)),
            pl.BlockSpec((1, NUM_EXPERTS), lambda i: (0, 0)),
        ],
        out_specs=(
            pl.BlockSpec((TOP_K, BLK), lambda i: (0, i)),
            pl.BlockSpec((TOP_K, BLK), lambda i: (0, i)),
            pl.BlockSpec((BLK, NUM_EXPERTS), lambda i: (i, 0)),
            pl.BlockSpec((1, 1), lambda i: (0, 0)),
        ),
        out_shape=out_shape,
        scratch_shapes=[pltpu.VMEM((1, NUM_EXPERTS), jnp.float32)],
    )(u, E, bias2)
    return topk_i_t.T, topk_s_t.T, scores, aux[0, 0]
